# Initial kernel scaffold; baseline (speedup 1.0000x reference)
import jax
import jax.numpy as jnp
from jax.experimental import pallas as pl


def kernel(x):
    raise NotImplementedError("write your pallas kernel here")


# speedup vs baseline: 11.9104x; 11.9104x over previous
import functools

import jax
import jax.numpy as jnp
from jax import lax
from jax.experimental import pallas as pl
from jax.experimental.pallas import tpu as pltpu
from jax.experimental.pallas import tpu_sc as plsc

NPOINT = 512
LANES = 16
BIG = 1e10


def _fps_body(xx_hbm, xy_hbm, xz_hbm, yx_hbm, yy_hbm, yz_hbm,
              xx_v, xy_v, xz_v, dst_v, pub_v, rd_v, idx_v,
              yx_s, yy_s, yz_s, shr):
    n = xx_v.shape[0]
    chunk_n = dst_v.shape[0]
    n_chunks = n // chunk_n
    n_slices = chunk_n // LANES

    c = lax.axis_index("c")
    sid = lax.axis_index("s")
    b_loc = sid // 8
    chunk = sid % 8
    batch = c * 2 + b_loc
    base = chunk * chunk_n

    pltpu.sync_copy(xx_hbm.at[batch], xx_v)
    pltpu.sync_copy(xy_hbm.at[batch], xy_v)
    pltpu.sync_copy(xz_hbm.at[batch], xz_v)

    def init_body(i, carry):
        dst_v[pl.ds(i * LANES, LANES)] = jnp.full((LANES,), BIG, jnp.float32)
        return carry
    lax.fori_loop(0, n_slices, init_body, 0)

    lanes = lax.broadcasted_iota(jnp.int32, (LANES,), 0)

    def step(s, w):
        cxv = plsc.load_gather(xx_v, [w])
        cyv = plsc.load_gather(xy_v, [w])
        czv = plsc.load_gather(xz_v, [w])

        row = (s // LANES) * LANES
        iv = idx_v[pl.ds(row, LANES)]
        idx_v[pl.ds(row, LANES)] = jnp.where(lanes == (s % LANES), w, iv)

        def inner(i, carry):
            maxv, maxi = carry
            off = base + i * LANES
            dx = xx_v[pl.ds(off, LANES)] - cxv
            dy = xy_v[pl.ds(off, LANES)] - cyv
            dz = xz_v[pl.ds(off, LANES)] - czv
            d = dx * dx + dy * dy + dz * dz
            nd = jnp.minimum(dst_v[pl.ds(i * LANES, LANES)], d)
            dst_v[pl.ds(i * LANES, LANES)] = nd
            upd = nd > maxv
            maxv = jnp.where(upd, nd, maxv)
            maxi = jnp.where(upd, jnp.full((LANES,), i, jnp.int32), maxi)
            return maxv, maxi

        maxv, maxi = lax.fori_loop(
            0, n_slices, inner,
            (jnp.full((LANES,), -1.0, jnp.float32),
             jnp.zeros((LANES,), jnp.int32)),
            unroll=4)

        m = jnp.max(maxv)
        mv = jnp.full((LANES,), m, jnp.float32)
        gidx = base + maxi * LANES + lanes
        cand = jnp.where(maxv == mv, gidx, jnp.int32(0x7FFFFFFF))
        li = jnp.min(cand)
        liv = jnp.full((LANES,), li, jnp.int32)

        pub_v[pl.ds(0, LANES)] = mv
        pub_v[pl.ds(LANES, LANES)] = plsc.bitcast(liv, jnp.float32)
        p = s % 2
        slot = (p * 2 + b_loc) * n_chunks + chunk
        pltpu.sync_copy(pub_v, shr.at[pl.ds(slot * 2 * LANES, 2 * LANES)])
        plsc.subcore_barrier()
        blk = (p * 2 + b_loc) * n_chunks * 2 * LANES
        pltpu.sync_copy(shr.at[pl.ds(blk, n_chunks * 2 * LANES)], rd_v)

        bv = rd_v[pl.ds(0, LANES)]
        bi = plsc.bitcast(rd_v[pl.ds(LANES, LANES)], jnp.int32)
        for j in range(1, n_chunks):
            vj = rd_v[pl.ds(j * 2 * LANES, LANES)]
            ij = plsc.bitcast(rd_v[pl.ds(j * 2 * LANES + LANES, LANES)],
                              jnp.int32)
            better = vj > bv
            bv = jnp.where(better, vj, bv)
            bi = jnp.where(better, ij, bi)
        return bi

    lax.fori_loop(0, NPOINT, step, jnp.zeros((LANES,), jnp.int32))

    @pl.when(chunk == 0)
    def _():
        for i in range(NPOINT // LANES):
            sl = pl.ds(i * LANES, LANES)
            ivec = idx_v[sl]
            yx_s[sl] = plsc.load_gather(xx_v, [ivec])
            yy_s[sl] = plsc.load_gather(xy_v, [ivec])
            yz_s[sl] = plsc.load_gather(xz_v, [ivec])
        pltpu.sync_copy(yx_s, yx_hbm.at[batch])
        pltpu.sync_copy(yy_s, yy_hbm.at[batch])
        pltpu.sync_copy(yz_s, yz_hbm.at[batch])


@functools.partial(jax.jit, static_argnums=())
def _fps_sc(xx, xy, xz):
    b, n = xx.shape
    out = jax.ShapeDtypeStruct((b, NPOINT), jnp.float32)
    fn = pl.kernel(
        _fps_body,
        out_type=(out, out, out),
        mesh=plsc.VectorSubcoreMesh(core_axis_name="c", subcore_axis_name="s",
                                    num_cores=2, num_subcores=16),
        compiler_params=pltpu.CompilerParams(needs_layout_passes=False),
        scratch_types=[
            pltpu.VMEM((n,), jnp.float32),
            pltpu.VMEM((n,), jnp.float32),
            pltpu.VMEM((n,), jnp.float32),
            pltpu.VMEM((n // 8,), jnp.float32),
            pltpu.VMEM((2 * LANES,), jnp.float32),
            pltpu.VMEM((8 * 2 * LANES,), jnp.float32),
            pltpu.VMEM((NPOINT,), jnp.int32),
            pltpu.VMEM((NPOINT,), jnp.float32),
            pltpu.VMEM((NPOINT,), jnp.float32),
            pltpu.VMEM((NPOINT,), jnp.float32),
            pltpu.VMEM_SHARED((2 * 2 * 8 * 2 * LANES,), jnp.float32),
        ],
    )
    return fn(xx, xy, xz)


def kernel(x):
    xx = x[:, :, 0]
    xy = x[:, :, 1]
    xz = x[:, :, 2]
    yx, yy, yz = _fps_sc(xx, xy, xz)
    return jnp.stack([yx, yy, yz], axis=-1)


# speedup vs baseline: 28.5935x; 2.4007x over previous
import functools

import jax
import jax.numpy as jnp
from jax import lax
from jax.experimental import pallas as pl
from jax.experimental.pallas import tpu as pltpu
from jax.experimental.pallas import tpu_sc as plsc

NPOINT = 512
LANES = 16
BIG = 1e10


def _fps_body(xx_hbm, xy_hbm, xz_hbm, yx_hbm, yy_hbm, yz_hbm,
              xx_v, xy_v, xz_v, dst_v, pub_v, rd_v, idx_v,
              yx_s, yy_s, yz_s, shr):
    n = xx_v.shape[0]
    chunk_n = dst_v.shape[0]
    n_chunks = n // chunk_n
    n_slices = chunk_n // LANES

    c = lax.axis_index("c")
    sid = lax.axis_index("s")
    b_loc = sid // 8
    chunk = sid % 8
    batch = c * 2 + b_loc
    base = chunk * chunk_n

    pltpu.sync_copy(xx_hbm.at[batch], xx_v)
    pltpu.sync_copy(xy_hbm.at[batch], xy_v)
    pltpu.sync_copy(xz_hbm.at[batch], xz_v)

    def init_body(i, carry):
        dst_v[pl.ds(i * LANES, LANES)] = jnp.full((LANES,), BIG, jnp.float32)
        return carry
    lax.fori_loop(0, n_slices, init_body, 0)

    lanes = lax.broadcasted_iota(jnp.int32, (LANES,), 0)

    def step(s, w):
        cxv = plsc.load_gather(xx_v, [w])
        cyv = plsc.load_gather(xy_v, [w])
        czv = plsc.load_gather(xz_v, [w])

        row = (s // LANES) * LANES
        iv = idx_v[pl.ds(row, LANES)]
        idx_v[pl.ds(row, LANES)] = jnp.where(lanes == (s % LANES), w, iv)

        ninf = jnp.full((LANES,), -1.0, jnp.float32)
        zi = jnp.zeros((LANES,), jnp.int32)

        def inner(i, carry):
            accs = list(carry)
            for j in range(4):
                off = base + i * LANES + j * LANES
                dx = xx_v[pl.ds(off, LANES)] - cxv
                dy = xy_v[pl.ds(off, LANES)] - cyv
                dz = xz_v[pl.ds(off, LANES)] - czv
                d = dx * dx + dy * dy + dz * dz
                doff = i * LANES + j * LANES
                nd = jnp.minimum(dst_v[pl.ds(doff, LANES)], d)
                dst_v[pl.ds(doff, LANES)] = nd
                av, ai = accs[2 * j], accs[2 * j + 1]
                upd = nd > av
                accs[2 * j] = jnp.where(upd, nd, av)
                accs[2 * j + 1] = jnp.where(
                    upd, jnp.full((LANES,), i + j, jnp.int32), ai)
            return tuple(accs)

        accs = plsc.parallel_loop(
            0, n_slices, step=4, unroll=2,
            carry=(ninf, zi, ninf, zi, ninf, zi, ninf, zi))(inner)

        maxv, maxi = accs[0], accs[1]
        for j in range(1, 4):
            av, ai = accs[2 * j], accs[2 * j + 1]
            upd = jnp.logical_or(
                av > maxv, jnp.logical_and(av == maxv, ai < maxi))
            maxv = jnp.where(upd, av, maxv)
            maxi = jnp.where(upd, ai, maxi)

        m = jnp.max(maxv)
        mv = jnp.full((LANES,), m, jnp.float32)
        gidx = base + maxi * LANES + lanes
        cand = jnp.where(maxv == mv, gidx, jnp.int32(0x7FFFFFFF))
        li = jnp.min(cand)
        liv = jnp.full((LANES,), li, jnp.int32)

        pub_v[pl.ds(0, LANES)] = mv
        pub_v[pl.ds(LANES, LANES)] = plsc.bitcast(liv, jnp.float32)
        p = s % 2
        slot = (p * 2 + b_loc) * n_chunks + chunk
        pltpu.sync_copy(pub_v, shr.at[pl.ds(slot * 2 * LANES, 2 * LANES)])
        plsc.subcore_barrier()
        blk = (p * 2 + b_loc) * n_chunks * 2 * LANES
        pltpu.sync_copy(shr.at[pl.ds(blk, n_chunks * 2 * LANES)], rd_v)

        bv = rd_v[pl.ds(0, LANES)]
        bi = plsc.bitcast(rd_v[pl.ds(LANES, LANES)], jnp.int32)
        for j in range(1, n_chunks):
            vj = rd_v[pl.ds(j * 2 * LANES, LANES)]
            ij = plsc.bitcast(rd_v[pl.ds(j * 2 * LANES + LANES, LANES)],
                              jnp.int32)
            better = vj > bv
            bv = jnp.where(better, vj, bv)
            bi = jnp.where(better, ij, bi)
        return bi

    lax.fori_loop(0, NPOINT, step, jnp.zeros((LANES,), jnp.int32))

    @pl.when(chunk == 0)
    def _():
        for i in range(NPOINT // LANES):
            sl = pl.ds(i * LANES, LANES)
            ivec = idx_v[sl]
            yx_s[sl] = plsc.load_gather(xx_v, [ivec])
            yy_s[sl] = plsc.load_gather(xy_v, [ivec])
            yz_s[sl] = plsc.load_gather(xz_v, [ivec])
        pltpu.sync_copy(yx_s, yx_hbm.at[batch])
        pltpu.sync_copy(yy_s, yy_hbm.at[batch])
        pltpu.sync_copy(yz_s, yz_hbm.at[batch])


@functools.partial(jax.jit, static_argnums=())
def _fps_sc(xx, xy, xz):
    b, n = xx.shape
    out = jax.ShapeDtypeStruct((b, NPOINT), jnp.float32)
    fn = pl.kernel(
        _fps_body,
        out_type=(out, out, out),
        mesh=plsc.VectorSubcoreMesh(core_axis_name="c", subcore_axis_name="s",
                                    num_cores=2, num_subcores=16),
        compiler_params=pltpu.CompilerParams(needs_layout_passes=False),
        scratch_types=[
            pltpu.VMEM((n,), jnp.float32),
            pltpu.VMEM((n,), jnp.float32),
            pltpu.VMEM((n,), jnp.float32),
            pltpu.VMEM((n // 8,), jnp.float32),
            pltpu.VMEM((2 * LANES,), jnp.float32),
            pltpu.VMEM((8 * 2 * LANES,), jnp.float32),
            pltpu.VMEM((NPOINT,), jnp.int32),
            pltpu.VMEM((NPOINT,), jnp.float32),
            pltpu.VMEM((NPOINT,), jnp.float32),
            pltpu.VMEM((NPOINT,), jnp.float32),
            pltpu.VMEM_SHARED((2 * 2 * 8 * 2 * LANES,), jnp.float32),
        ],
    )
    return fn(xx, xy, xz)


def kernel(x):
    xx = x[:, :, 0]
    xy = x[:, :, 1]
    xz = x[:, :, 2]
    yx, yy, yz = _fps_sc(xx, xy, xz)
    return jnp.stack([yx, yy, yz], axis=-1)
